# Initial kernel scaffold; baseline (speedup 1.0000x reference)
import jax
import jax.numpy as jnp
from jax.experimental import pallas as pl


def kernel(x, table):
    raise NotImplementedError("write your pallas kernel here")


# speedup vs baseline: 1.4944x; 1.4944x over previous
import functools
import math

import jax
import jax.numpy as jnp
from jax import lax
from jax.experimental import pallas as pl
from jax.experimental.pallas import tpu as pltpu
from jax.experimental.pallas import tpu_sc as plsc

D_MODEL = 1024
SCALE = math.sqrt(D_MODEL)
LANES = 16
NW = 32
CH = 16
NBUF = 4
LOOKAHEAD = 2


def _embed(idx, table):
    (B,) = idx.shape
    V, D = table.shape
    b_per_w = B // NW
    n_ch = b_per_w // CH
    n_grp = n_ch // NBUF
    vecs_per_chunk = CH * D // LANES

    mesh = plsc.VectorSubcoreMesh(core_axis_name="c", subcore_axis_name="s")

    @functools.partial(
        pl.kernel,
        out_type=jax.ShapeDtypeStruct((B, D), jnp.float32),
        mesh=mesh,
        scratch_types=[
            pltpu.VMEM((b_per_w,), jnp.int32),
            pltpu.VMEM((NBUF, CH, D), jnp.float32),
            pltpu.SemaphoreType.DMA((NBUF,)),
            pltpu.SemaphoreType.DMA((NBUF,)),
        ],
    )
    def emb(table_hbm, idx_hbm, out_hbm, idx_v, bufs, gsems, osems):
        wid = lax.axis_index("s") * 2 + lax.axis_index("c")
        base = wid * b_per_w
        pltpu.sync_copy(idx_hbm.at[pl.ds(base, b_per_w)], idx_v)

        def gather_start(c, j):
            pltpu.async_copy(
                table_hbm.at[idx_v.at[pl.ds(c * CH, CH)]],
                bufs.at[j],
                gsems.at[j],
            )

        def gather_wait(j):
            pltpu.make_async_copy(
                table_hbm.at[pl.ds(0, CH)], bufs.at[j], gsems.at[j]
            ).wait()

        def wb_start(c, j):
            pltpu.async_copy(
                bufs.at[j], out_hbm.at[pl.ds(base + c * CH, CH)], osems.at[j]
            )

        def wb_wait(j):
            pltpu.make_async_copy(
                bufs.at[j], out_hbm.at[pl.ds(0, CH)], osems.at[j]
            ).wait()

        def scale_buf(j):
            b = bufs.at[j]

            @plsc.parallel_loop(0, CH)
            def _(r):
                for k in range(D // LANES):
                    sl = pl.ds(k * LANES, LANES)
                    b[r, sl] = b[r, sl] * SCALE

        def step(c, jj, wb_pending):
            jg = (jj + LOOKAHEAD) % NBUF
            if wb_pending:
                wb_wait(jg)
            gather_start(c + LOOKAHEAD, jg)
            gather_wait(jj)
            scale_buf(jj)
            wb_start(c, jj)

        for c0 in range(LOOKAHEAD):
            gather_start(c0, c0 % NBUF)

        for jj in range(NBUF):
            step(jj, jj, wb_pending=jj + LOOKAHEAD >= NBUF)

        def body(g, _):
            c_base = g * NBUF
            for jj in range(NBUF):
                step(c_base + jj, jj, wb_pending=True)
            return 0

        lax.fori_loop(1, n_grp - 1, body, 0)

        c_base = (n_grp - 1) * NBUF
        for jj in range(NBUF):
            c = c_base + jj
            if jj + LOOKAHEAD < NBUF:
                jg = (jj + LOOKAHEAD) % NBUF
                wb_wait(jg)
                gather_start(c + LOOKAHEAD, jg)
            gather_wait(jj)
            scale_buf(jj)
            wb_start(c, jj)

        for jj in range(NBUF):
            wb_wait(jj)

    return emb(table, idx)


def kernel(x, table):
    B0, S = x.shape
    idx = x.reshape(B0 * S).astype(jnp.int32)
    out = _embed(idx, table)
    return out.reshape(B0, S, table.shape[1])
